# Initial kernel scaffold; baseline (speedup 1.0000x reference)
import jax
import jax.numpy as jnp
from jax.experimental import pallas as pl


def kernel(targets):
    raise NotImplementedError("write your pallas kernel here")


# speedup vs baseline: 8.2660x; 8.2660x over previous
import functools

import jax
import jax.numpy as jnp
import numpy as np
from jax.experimental import pallas as pl
from jax.experimental.pallas import tpu as pltpu

_VOCAB_SIZE = 32000
_PADDING_IDX = 0
_SMOOTHING = 0.1
_CONFIDENCE = np.float32(1.0 - _SMOOTHING)
_SMOOTH_VAL = np.float32(_SMOOTHING / (_VOCAB_SIZE - 2))

_ROW_BLK = 64


def _fill_body(tgt_ref, out_ref):
    t = tgt_ref[0, 0, :]
    tcol = t[:, None]
    vocab = jax.lax.broadcasted_iota(jnp.int32, (_ROW_BLK, _VOCAB_SIZE), 1)
    val = jnp.where(vocab == tcol, _CONFIDENCE, _SMOOTH_VAL)
    val = jnp.where((vocab == _PADDING_IDX) | (tcol == _PADDING_IDX),
                    jnp.float32(0.0), val)
    out_ref[...] = val


@jax.jit
def kernel(targets):
    batch_size, tgt_seq_len = targets.shape
    rows = batch_size * tgt_seq_len
    num_blocks = rows // _ROW_BLK
    tgt_r = targets.reshape(num_blocks, 1, _ROW_BLK)

    out = pl.pallas_call(
        _fill_body,
        grid=(num_blocks,),
        in_specs=[pl.BlockSpec((1, 1, _ROW_BLK), lambda i: (i, 0, 0))],
        out_specs=pl.BlockSpec((_ROW_BLK, _VOCAB_SIZE), lambda i: (i, 0)),
        out_shape=jax.ShapeDtypeStruct((rows, _VOCAB_SIZE), jnp.float32),
    )(tgt_r)
    return out.reshape(batch_size, tgt_seq_len, _VOCAB_SIZE)
